# baseline (speedup 1.0000x reference)
import functools

import jax
import jax.numpy as jnp
from jax import lax
from jax.experimental import pallas as pl
from jax.experimental.pallas import tpu as pltpu
from jax.experimental.pallas import tpu_sc as plsc

SEQ_LEN = 200
BATCH = 1024
D_MODEL = 64
RG = 8
CG = 4
RPW = SEQ_LEN // RG
CPW = BATCH // CG
CH = 128
NBUF = 5
NGRP = RPW // NBUF

_mesh = plsc.VectorSubcoreMesh(core_axis_name="c", subcore_axis_name="s")


@functools.partial(
    pl.kernel,
    mesh=_mesh,
    out_type=jax.ShapeDtypeStruct((SEQ_LEN, BATCH, D_MODEL), jnp.float32),
    scratch_types=[
        pltpu.VMEM((RPW, CPW), jnp.int32),
        pltpu.VMEM((NBUF, CPW, D_MODEL), jnp.float32),
        pltpu.SemaphoreType.DMA,
    ],
    compiler_params=pltpu.CompilerParams(use_tc_tiling_on_sc=False),
)
def _gather(table_hbm, idx_hbm, out_hbm, idx_v, rows_v, gsem):
    wid = lax.axis_index("s") * 2 + lax.axis_index("c")
    rg = wid // CG
    cg = wid % CG
    s_base = rg * RPW
    c_base = cg * CPW
    pltpu.sync_copy(
        idx_hbm.at[pl.ds(s_base, RPW), pl.ds(c_base, CPW)], idx_v
    )

    def start(r, buf):
        for h in range(CPW // CH):
            pltpu.async_copy(
                table_hbm.at[idx_v.at[r, pl.ds(h * CH, CH)]],
                rows_v.at[buf, pl.ds(h * CH, CH)],
                gsem,
            )

    def finish(r, buf):
        for h in range(CPW // CH):
            pltpu.make_async_copy(
                table_hbm.at[idx_v.at[r, pl.ds(h * CH, CH)]],
                rows_v.at[buf, pl.ds(h * CH, CH)],
                gsem,
            ).wait()
        pltpu.sync_copy(
            rows_v.at[buf], out_hbm.at[s_base + r, pl.ds(c_base, CPW)]
        )

    for b in range(NBUF):
        start(b, b)

    def body(g, carry):
        for b in range(NBUF):
            r = g * NBUF + b
            finish(r, b)
            start(r + NBUF, b)
        return carry

    lax.fori_loop(0, NGRP - 1, body, 0)

    for b in range(NBUF):
        finish((NGRP - 1) * NBUF + b, b)


def kernel(input, table):
    return _gather(table, input)


# speedup vs baseline: 1.1437x; 1.1437x over previous
import functools

import jax
import jax.numpy as jnp
from jax import lax
from jax.experimental import pallas as pl
from jax.experimental.pallas import tpu as pltpu
from jax.experimental.pallas import tpu_sc as plsc

SEQ_LEN = 200
BATCH = 1024
D_MODEL = 64
DP = 128
NW = 32
TS = 8
TL = 128
NTS = SEQ_LEN // TS
NTL = BATCH // TL
NT = NTS * NTL
KMAX = -(-NT // NW)

_mesh = plsc.VectorSubcoreMesh(core_axis_name="c", subcore_axis_name="s")


@functools.partial(
    pl.kernel,
    mesh=_mesh,
    out_type=jax.ShapeDtypeStruct((SEQ_LEN, BATCH, DP), jnp.float32),
    scratch_types=[
        pltpu.VMEM((TS, TL), jnp.int32),
        pltpu.VMEM((4 * TL, DP), jnp.float32),
        pltpu.SemaphoreType.DMA,
    ],
)
def _gather(table_hbm, idx_hbm, out_hbm, idx_v, rows_v, gsem):
    wid = lax.axis_index("s") * 2 + lax.axis_index("c")

    def do_tile(t):
        ts = t // NTL
        tl = t % NTL
        pltpu.sync_copy(
            idx_hbm.at[pl.ds(ts * TS, TS), pl.ds(tl * TL, TL)], idx_v
        )
        for half in range(2):
            for j in range(4):
                pltpu.async_copy(
                    table_hbm.at[idx_v.at[half * 4 + j]],
                    rows_v.at[pl.ds(j * TL, TL)],
                    gsem,
                )
            for j in range(4):
                pltpu.make_async_copy(
                    table_hbm.at[idx_v.at[half * 4 + j]],
                    rows_v.at[pl.ds(j * TL, TL)],
                    gsem,
                ).wait()
                s = ts * TS + half * 4 + j
                pltpu.sync_copy(
                    rows_v.at[pl.ds(j * TL, TL)],
                    out_hbm.at[s, pl.ds(tl * TL, TL)],
                )

    for k in range(KMAX):
        t = wid + k * NW

        @pl.when(t < NT)
        def _():
            do_tile(t)


def kernel(input, table):
    table128 = jnp.pad(table, ((0, 0), (0, DP - D_MODEL)))
    out128 = _gather(table128, input)
    return out128[:, :, :D_MODEL]


# speedup vs baseline: 1.1551x; 1.0099x over previous
import functools

import jax
import jax.numpy as jnp
from jax import lax
from jax.experimental import pallas as pl
from jax.experimental.pallas import tpu as pltpu
from jax.experimental.pallas import tpu_sc as plsc

SEQ_LEN = 200
BATCH = 1024
D_MODEL = 64
DP = 128
NW = 32
TS = 8
TL = 128
NTL = BATCH // TL
NT = (SEQ_LEN // TS) * NTL
KMAX = -(-NT // NW)
NCH = KMAX * TS
NBUF = 6

_mesh = plsc.VectorSubcoreMesh(core_axis_name="c", subcore_axis_name="s")


@functools.partial(
    pl.kernel,
    mesh=_mesh,
    out_type=jax.ShapeDtypeStruct((SEQ_LEN, BATCH, DP), jnp.float32),
    scratch_types=[
        pltpu.VMEM((2, TS, TL), jnp.int32),
        pltpu.VMEM((NBUF, TL, DP), jnp.float32),
        pltpu.SemaphoreType.DMA,
    ],
)
def _gather(table_hbm, idx_hbm, out_hbm, idx_v, rows_v, gsem):
    wid = lax.axis_index("s") * 2 + lax.axis_index("c")

    def load_idx(k):
        t = wid + k * NW

        @pl.when(t < NT)
        def _():
            ts = t // NTL
            tl = t % NTL
            pltpu.sync_copy(
                idx_hbm.at[pl.ds(ts * TS, TS), pl.ds(tl * TL, TL)],
                idx_v.at[lax.rem(k, 2)],
            )

    def start(q):
        k = q // TS
        t = wid + k * NW

        @pl.when(t < NT)
        def _():
            pltpu.async_copy(
                table_hbm.at[idx_v.at[lax.rem(k, 2), lax.rem(q, TS)]],
                rows_v.at[lax.rem(q, NBUF)],
                gsem,
            )

    def finish(q):
        k = q // TS
        t = wid + k * NW

        @pl.when(t < NT)
        def _():
            ts = t // NTL
            tl = t % NTL
            pltpu.make_async_copy(
                table_hbm.at[idx_v.at[lax.rem(k, 2), lax.rem(q, TS)]],
                rows_v.at[lax.rem(q, NBUF)],
                gsem,
            ).wait()
            pltpu.sync_copy(
                rows_v.at[lax.rem(q, NBUF)],
                out_hbm.at[ts * TS + lax.rem(q, TS), pl.ds(tl * TL, TL)],
            )

    load_idx(0)
    for q in range(NBUF):
        start(q)

    def body(g, carry):
        nxt = g + NBUF

        @pl.when(lax.rem(nxt, TS) == 0)
        def _():
            load_idx(nxt // TS)

        finish(g)
        start(nxt)
        return carry

    lax.fori_loop(0, NCH - NBUF, body, 0)

    for q in range(NCH - NBUF, NCH):
        finish(q)


def kernel(input, table):
    table128 = jnp.pad(table, ((0, 0), (0, DP - D_MODEL)))
    out128 = _gather(table128, input)
    return out128[:, :, :D_MODEL]


# speedup vs baseline: 1.1583x; 1.0028x over previous
import functools

import jax
import jax.numpy as jnp
from jax import lax
from jax.experimental import pallas as pl
from jax.experimental.pallas import tpu as pltpu
from jax.experimental.pallas import tpu_sc as plsc

SEQ_LEN = 200
BATCH = 1024
D_MODEL = 64
DP = 128
NW = 32
TS = 8
TL = 128
NTL = BATCH // TL
NT = (SEQ_LEN // TS) * NTL
KMAX = -(-NT // NW)
NCH = KMAX * TS
NBUF = 7

_mesh = plsc.VectorSubcoreMesh(core_axis_name="c", subcore_axis_name="s")


@functools.partial(
    pl.kernel,
    mesh=_mesh,
    out_type=jax.ShapeDtypeStruct((SEQ_LEN, BATCH, DP), jnp.float32),
    scratch_types=[
        pltpu.VMEM((2, TS, TL), jnp.int32),
        pltpu.VMEM((NBUF, TL, DP), jnp.float32),
        pltpu.SemaphoreType.DMA,
    ],
)
def _gather(table_hbm, idx_hbm, out_hbm, idx_v, rows_v, gsem):
    wid = lax.axis_index("s") * 2 + lax.axis_index("c")

    def load_idx(k):
        t = wid + k * NW

        @pl.when(t < NT)
        def _():
            ts = t // NTL
            tl = t % NTL
            pltpu.sync_copy(
                idx_hbm.at[pl.ds(ts * TS, TS), pl.ds(tl * TL, TL)],
                idx_v.at[lax.rem(k, 2)],
            )

    def start(q):
        k = q // TS
        t = wid + k * NW

        @pl.when(t < NT)
        def _():
            pltpu.async_copy(
                table_hbm.at[idx_v.at[lax.rem(k, 2), lax.rem(q, TS)]],
                rows_v.at[lax.rem(q, NBUF)],
                gsem,
            )

    def finish(q):
        k = q // TS
        t = wid + k * NW

        @pl.when(t < NT)
        def _():
            ts = t // NTL
            tl = t % NTL
            pltpu.make_async_copy(
                table_hbm.at[idx_v.at[lax.rem(k, 2), lax.rem(q, TS)]],
                rows_v.at[lax.rem(q, NBUF)],
                gsem,
            ).wait()
            pltpu.sync_copy(
                rows_v.at[lax.rem(q, NBUF)],
                out_hbm.at[ts * TS + lax.rem(q, TS), pl.ds(tl * TL, TL)],
            )

    load_idx(0)
    for q in range(NBUF):
        start(q)

    def body(g, carry):
        nxt = g + NBUF

        @pl.when(lax.rem(nxt, TS) == 0)
        def _():
            load_idx(nxt // TS)

        finish(g)
        start(nxt)
        return carry

    lax.fori_loop(0, NCH - NBUF, body, 0)

    for q in range(NCH - NBUF, NCH):
        finish(q)


def kernel(input, table):
    table128 = jnp.pad(table, ((0, 0), (0, DP - D_MODEL)))
    out128 = _gather(table128, input)
    return out128[:, :, :D_MODEL]
